# baseline (speedup 1.0000x reference)
import functools
import math

import jax
import jax.numpy as jnp
from jax import lax
from jax.experimental import pallas as pl
from jax.experimental.pallas import tpu as pltpu
from jax.experimental.pallas import tpu_sc as plsc

N_Q = 10000
N_S = 10000
NN = 32
IN_DIM = 128
OUT_DIM = 128
KL = 27
RADIUS = 0.2
KP_EXTENT = 2.0 * RADIUS / 2.0 / math.sqrt(3.0)
ROWS = N_Q * NN

_NC = 2
_NSC = 16
_NW = _NC * _NSC
_GROUP_IR = 3
_CH = _GROUP_IR * 128
_NITER = 27
ROWS_PAD = _NW * _NITER * _CH


_NS_PAD = 10008


def _sc_gather_body(idx_hbm, xpad_hbm, sx_hbm, sy_hbm, sz_hbm, zz_hbm,
                    xg_out, pos_out,
                    idx_v, xrows_v, prows_v, sx_v, sy_v, sz_v, semx):
    wid = lax.axis_index("s") * _NC + lax.axis_index("c")
    pltpu.sync_copy(sx_hbm, sx_v)
    pltpu.sync_copy(sy_hbm, sy_v)
    pltpu.sync_copy(sz_hbm, sz_v)
    pltpu.sync_copy(zz_hbm, prows_v)

    def body(t, carry):
        g = wid + _NW * t
        pltpu.sync_copy(idx_hbm.at[pl.ds(g * _CH, _CH)], idx_v)
        copies = []
        for b in range(_GROUP_IR):
            copies.append(pltpu.async_copy(
                xpad_hbm.at[idx_v.at[pl.ds(b * 128, 128)]],
                xrows_v.at[pl.ds(b * 128, 128)], semx))
        lanes = lax.iota(jnp.int32, 16)
        for j in range(_CH // 16):
            ii = idx_v[pl.ds(j * 16, 16)]
            rows = lanes + (j * 16)
            px = plsc.load_gather(sx_v, [ii])
            py = plsc.load_gather(sy_v, [ii])
            pz = plsc.load_gather(sz_v, [ii])
            plsc.store_scatter(prows_v, [rows, jnp.full((16,), 0, jnp.int32)], px)
            plsc.store_scatter(prows_v, [rows, jnp.full((16,), 1, jnp.int32)], py)
            plsc.store_scatter(prows_v, [rows, jnp.full((16,), 2, jnp.int32)], pz)
        for cp in copies:
            cp.wait()
        base = g * _CH
        pltpu.sync_copy(xrows_v, xg_out.at[pl.ds(base, _CH)])
        pltpu.sync_copy(prows_v, pos_out.at[pl.ds(base, _CH)])
        return carry

    lax.fori_loop(0, _NITER, body, 0)


@functools.lru_cache(maxsize=1)
def _sc_gather():
    return pl.kernel(
        _sc_gather_body,
        out_type=(jax.ShapeDtypeStruct((ROWS_PAD, IN_DIM), jnp.float32),
                  jax.ShapeDtypeStruct((ROWS_PAD, 16), jnp.float32)),
        mesh=plsc.VectorSubcoreMesh(core_axis_name="c", subcore_axis_name="s"),
        compiler_params=pltpu.CompilerParams(needs_layout_passes=False),
        scratch_types=[
            pltpu.VMEM((_CH,), jnp.int32),
            pltpu.VMEM((_CH, IN_DIM), jnp.float32),
            pltpu.VMEM((_CH, 16), jnp.float32),
            pltpu.VMEM((_NS_PAD,), jnp.float32),
            pltpu.VMEM((_NS_PAD,), jnp.float32),
            pltpu.VMEM((_NS_PAD,), jnp.float32),
            pltpu.SemaphoreType.DMA,
        ],
    )


_QB = 200
_PB = _QB * NN
_NBLK = N_Q // _QB


def _conv_body(qrep_ref, pos_ref, xg_ref, kpt_ref, w_ref, out_ref):
    rel = pos_ref[...] - qrep_ref[...]
    kpt = kpt_ref[...]
    dots = jnp.dot(rel, kpt, preferred_element_type=jnp.float32,
                   precision=jax.lax.Precision.HIGHEST)
    kn = jnp.sum(kpt * kpt, axis=0, keepdims=True)
    sqn = jnp.sum(rel * rel, axis=1, keepdims=True)
    sq = jnp.maximum(sqn + kn - 2.0 * dots, 0.0)
    aw = jnp.maximum(1.0 - jnp.sqrt(sq) * (1.0 / KP_EXTENT), 0.0)
    aw3 = aw.reshape(_QB, NN, 32)
    xg = xg_ref[...]
    acc = jnp.zeros((_QB, OUT_DIM), dtype=jnp.float32)
    for l in range(KL):
        zl = aw3[:, :, l:l + 1] * xg
        zsum = jnp.sum(zl, axis=1)
        acc = acc + jnp.dot(zsum, w_ref[l],
                            preferred_element_type=jnp.float32)
    out_ref[...] = jnp.maximum(acc, 0.0)


def _conv_call(qrep, pos, xg3, kpt, weights, interpret=False):
    return pl.pallas_call(
        _conv_body,
        grid=(_NBLK,),
        in_specs=[
            pl.BlockSpec((_PB, 16), lambda i: (i, 0)),
            pl.BlockSpec((_PB, 16), lambda i: (i, 0)),
            pl.BlockSpec((_QB, NN, IN_DIM), lambda i: (i, 0, 0)),
            pl.BlockSpec((16, 32), lambda i: (0, 0)),
            pl.BlockSpec((KL, IN_DIM, OUT_DIM), lambda i: (0, 0, 0)),
        ],
        out_specs=pl.BlockSpec((_QB, OUT_DIM), lambda i: (i, 0)),
        out_shape=jax.ShapeDtypeStruct((N_Q, OUT_DIM), jnp.float32),
        compiler_params=pltpu.CompilerParams(
            dimension_semantics=("arbitrary",)),
        interpret=interpret,
    )(qrep, pos, xg3, kpt, weights)


def kernel(q_pts, s_pts, neighb_inds, x, weights, kernel_points):
    idx = (neighb_inds.astype(jnp.int32) % (N_S + 1)).reshape(-1)
    idx_pad = jnp.concatenate(
        [idx, jnp.zeros((ROWS_PAD - ROWS,), jnp.int32)])
    x_pad = jnp.concatenate([x, jnp.zeros((1, IN_DIM), x.dtype)], axis=0)
    tail = jnp.concatenate([jnp.full((1,), 1.0e6, s_pts.dtype),
                            jnp.zeros((_NS_PAD - N_S - 1,), s_pts.dtype)])
    sx = jnp.concatenate([s_pts[:, 0], tail])
    sy = jnp.concatenate([s_pts[:, 1], tail])
    sz = jnp.concatenate([s_pts[:, 2], tail])
    zz = jnp.zeros((_CH, 16), jnp.float32)
    q16 = jnp.pad(q_pts, ((0, 0), (0, 13)))
    qrep = jnp.broadcast_to(q16[:, None, :], (N_Q, NN, 16)).reshape(ROWS, 16)
    kpt = jnp.pad(kernel_points, ((0, 5), (0, 13))).T

    xg, pos = _sc_gather()(idx_pad, x_pad, sx, sy, sz, zz)
    xg3 = xg.reshape(ROWS_PAD // NN, NN, IN_DIM)
    return _conv_call(qrep, pos, xg3, kpt, weights)


# speedup vs baseline: 1.2839x; 1.2839x over previous
import functools
import math

import jax
import jax.numpy as jnp
from jax import lax
from jax.experimental import pallas as pl
from jax.experimental.pallas import tpu as pltpu
from jax.experimental.pallas import tpu_sc as plsc

N_Q = 10000
N_S = 10000
NN = 32
IN_DIM = 128
OUT_DIM = 128
KL = 27
RADIUS = 0.2
KP_EXTENT = 2.0 * RADIUS / 2.0 / math.sqrt(3.0)
ROWS = N_Q * NN

_NC = 2
_NSC = 16
_NW = _NC * _NSC
_CH = 192
_SPLITS = ((0, 128), (128, 64))
_NITER = 54
ROWS_PAD = _NW * _NITER * _CH
_NS_PAD = 10008


def _sc_gather_body(idx_hbm, xpad_hbm, sx_hbm, sy_hbm, sz_hbm, zz_hbm,
                    xg_out, pos_out,
                    idx0, idx1, xr0, xr1, pr0, pr1, sx_v, sy_v, sz_v,
                    sg0, sg1, sw0, sw1):
    wid = lax.axis_index("s") * _NC + lax.axis_index("c")
    pltpu.sync_copy(sx_hbm, sx_v)
    pltpu.sync_copy(sy_hbm, sy_v)
    pltpu.sync_copy(sz_hbm, sz_v)
    pltpu.sync_copy(zz_hbm, pr0)
    pltpu.sync_copy(zz_hbm, pr1)

    def start(c, idx_b, xr_b, sg):
        pltpu.sync_copy(idx_hbm.at[pl.ds(c * _CH, _CH)], idx_b)
        for off, ln in _SPLITS:
            pltpu.async_copy(
                xpad_hbm.at[idx_b.at[pl.ds(off, ln)]],
                xr_b.at[pl.ds(off, ln)], sg)

    def assemble_pos(idx_b, pr_b):
        lanes = lax.iota(jnp.int32, 16)
        for j in range(_CH // 16):
            ii = idx_b[pl.ds(j * 16, 16)]
            rows = lanes + (j * 16)
            px = plsc.load_gather(sx_v, [ii])
            py = plsc.load_gather(sy_v, [ii])
            pz = plsc.load_gather(sz_v, [ii])
            plsc.store_scatter(pr_b, [rows, jnp.full((16,), 0, jnp.int32)], px)
            plsc.store_scatter(pr_b, [rows, jnp.full((16,), 1, jnp.int32)], py)
            plsc.store_scatter(pr_b, [rows, jnp.full((16,), 2, jnp.int32)], pz)

    def drain_gather(xr_b, sg):
        for off, ln in _SPLITS:
            pltpu.make_async_copy(
                xpad_hbm.at[idx0.at[pl.ds(off, ln)]],
                xr_b.at[pl.ds(off, ln)], sg).wait()

    def fire_writeout(c, xr_b, pr_b, sw):
        base = c * _CH
        pltpu.async_copy(xr_b, xg_out.at[pl.ds(base, _CH)], sw)
        pltpu.async_copy(pr_b, pos_out.at[pl.ds(base, _CH)], sw)

    def drain_writeout(xr_b, pr_b, sw):
        pltpu.make_async_copy(xr_b, xg_out.at[pl.ds(0, _CH)], sw).wait()
        pltpu.make_async_copy(pr_b, pos_out.at[pl.ds(0, _CH)], sw).wait()

    def body(j, carry):
        c0 = wid + _NW * (2 * j)
        c1 = wid + _NW * (2 * j + 1)

        @pl.when(j > 0)
        def _():
            drain_writeout(xr0, pr0, sw0)
        start(c0, idx0, xr0, sg0)

        @pl.when(j > 0)
        def _():
            drain_writeout(xr1, pr1, sw1)
        start(c1, idx1, xr1, sg1)

        assemble_pos(idx0, pr0)
        drain_gather(xr0, sg0)
        fire_writeout(c0, xr0, pr0, sw0)

        assemble_pos(idx1, pr1)
        drain_gather(xr1, sg1)
        fire_writeout(c1, xr1, pr1, sw1)
        return carry

    lax.fori_loop(0, _NITER // 2, body, 0)
    drain_writeout(xr0, pr0, sw0)
    drain_writeout(xr1, pr1, sw1)


@functools.lru_cache(maxsize=1)
def _sc_gather():
    return pl.kernel(
        _sc_gather_body,
        out_type=(jax.ShapeDtypeStruct((ROWS_PAD, IN_DIM), jnp.float32),
                  jax.ShapeDtypeStruct((ROWS_PAD, 16), jnp.float32)),
        mesh=plsc.VectorSubcoreMesh(core_axis_name="c", subcore_axis_name="s"),
        compiler_params=pltpu.CompilerParams(needs_layout_passes=False),
        scratch_types=[
            pltpu.VMEM((_CH,), jnp.int32),
            pltpu.VMEM((_CH,), jnp.int32),
            pltpu.VMEM((_CH, IN_DIM), jnp.float32),
            pltpu.VMEM((_CH, IN_DIM), jnp.float32),
            pltpu.VMEM((_CH, 16), jnp.float32),
            pltpu.VMEM((_CH, 16), jnp.float32),
            pltpu.VMEM((_NS_PAD,), jnp.float32),
            pltpu.VMEM((_NS_PAD,), jnp.float32),
            pltpu.VMEM((_NS_PAD,), jnp.float32),
            pltpu.SemaphoreType.DMA,
            pltpu.SemaphoreType.DMA,
            pltpu.SemaphoreType.DMA,
            pltpu.SemaphoreType.DMA,
        ],
    )


_QB = 200
_PB = _QB * NN
_NBLK = N_Q // _QB


def _conv_body(qrep_ref, pos_ref, xg_ref, kpt_ref, w_ref, out_ref):
    rel = pos_ref[...] - qrep_ref[...]
    kpt = kpt_ref[...]
    dots = jnp.dot(rel, kpt, preferred_element_type=jnp.float32,
                   precision=jax.lax.Precision.HIGHEST)
    kn = jnp.sum(kpt * kpt, axis=0, keepdims=True)
    sqn = jnp.sum(rel * rel, axis=1, keepdims=True)
    sq = jnp.maximum(sqn + kn - 2.0 * dots, 0.0)
    aw = jnp.maximum(1.0 - jnp.sqrt(sq) * (1.0 / KP_EXTENT), 0.0)
    aw3 = aw.astype(jnp.bfloat16).reshape(_QB, NN, 32)
    xg = xg_ref[...].astype(jnp.bfloat16)
    acc = jnp.zeros((_QB, OUT_DIM), dtype=jnp.float32)
    for l in range(KL):
        zl = aw3[:, :, l:l + 1] * xg
        zsum = jnp.sum(zl, axis=1)
        acc = acc + jnp.dot(zsum, w_ref[l],
                            preferred_element_type=jnp.float32)
    out_ref[...] = jnp.maximum(acc, 0.0)


def _conv_call(qrep, pos, xg3, kpt, weights, interpret=False):
    return pl.pallas_call(
        _conv_body,
        grid=(_NBLK,),
        in_specs=[
            pl.BlockSpec((_PB, 16), lambda i: (i, 0)),
            pl.BlockSpec((_PB, 16), lambda i: (i, 0)),
            pl.BlockSpec((_QB, NN, IN_DIM), lambda i: (i, 0, 0)),
            pl.BlockSpec((16, 32), lambda i: (0, 0)),
            pl.BlockSpec((KL, IN_DIM, OUT_DIM), lambda i: (0, 0, 0)),
        ],
        out_specs=pl.BlockSpec((_QB, OUT_DIM), lambda i: (i, 0)),
        out_shape=jax.ShapeDtypeStruct((N_Q, OUT_DIM), jnp.float32),
        compiler_params=pltpu.CompilerParams(
            dimension_semantics=("arbitrary",)),
        interpret=interpret,
    )(qrep, pos, xg3, kpt, weights)


def kernel(q_pts, s_pts, neighb_inds, x, weights, kernel_points):
    idx = (neighb_inds.astype(jnp.int32) % (N_S + 1)).reshape(-1)
    idx_pad = jnp.concatenate(
        [idx, jnp.zeros((ROWS_PAD - ROWS,), jnp.int32)])
    x_pad = jnp.concatenate([x, jnp.zeros((1, IN_DIM), x.dtype)], axis=0)
    tail = jnp.concatenate([jnp.full((1,), 1.0e6, s_pts.dtype),
                            jnp.zeros((_NS_PAD - N_S - 1,), s_pts.dtype)])
    sx = jnp.concatenate([s_pts[:, 0], tail])
    sy = jnp.concatenate([s_pts[:, 1], tail])
    sz = jnp.concatenate([s_pts[:, 2], tail])
    zz = jnp.zeros((_CH, 16), jnp.float32)
    q16 = jnp.pad(q_pts, ((0, 0), (0, 13)))
    qrep = jnp.broadcast_to(q16[:, None, :], (N_Q, NN, 16)).reshape(ROWS, 16)
    kpt = jnp.pad(kernel_points, ((0, 5), (0, 13))).T
    w_bf = weights.astype(jnp.bfloat16)

    xg, pos = _sc_gather()(idx_pad, x_pad, sx, sy, sz, zz)
    xg3 = xg.reshape(ROWS_PAD // NN, NN, IN_DIM)
    return _conv_call(qrep, pos, xg3, kpt, w_bf)
